# Initial kernel scaffold; baseline (speedup 1.0000x reference)
import jax
import jax.numpy as jnp
from jax.experimental import pallas as pl


def kernel(x, edge_index, W1, b1, W2, b2):
    raise NotImplementedError("write your pallas kernel here")


# speedup vs baseline: 10.9001x; 10.9001x over previous
import functools

import jax
import jax.numpy as jnp
from jax import lax
from jax.experimental import pallas as pl
from jax.experimental.pallas import tpu as pltpu
from jax.experimental.pallas import tpu_sc as plsc

N = 10000
NP = 10240
B = 4
F = 128
E = 320000
CHUNK = 128
NSUB = 16
G = 16
NG = 10
K = G * NG
EP = NSUB * CHUNK * K
RPT = NP // NSUB

_mesh = plsc.VectorSubcoreMesh(core_axis_name="c", subcore_axis_name="s",
                               num_cores=2, num_subcores=NSUB)


@functools.partial(
    pl.kernel,
    out_type=jax.ShapeDtypeStruct((2 * NP, F), jnp.float32),
    mesh=_mesh,
    scratch_types=[
        pltpu.VMEM_SHARED((NP, F), jnp.float32),
        pltpu.VMEM((G, CHUNK), jnp.int32),
        pltpu.VMEM((CHUNK, F), jnp.float32),
    ],
)
def _deg_kernel(dstl_hbm, ones_hbm, zeros_hbm, out_hbm, acc, didx, ones_v):
    c = lax.axis_index("c")
    s = lax.axis_index("s")
    pltpu.sync_copy(ones_hbm, ones_v)
    pltpu.sync_copy(zeros_hbm, acc.at[pl.ds(s * RPT, RPT)])
    plsc.subcore_barrier()

    def group(g, carry):
        pltpu.sync_copy(dstl_hbm.at[s, pl.ds(g * G, G)], didx)

        def chunk(k, carry2):
            pltpu.sync_copy(ones_v, acc.at[didx.at[k]], add=True)
            return carry2

        return lax.fori_loop(0, G, chunk, carry)

    lax.fori_loop(c * (NG // 2), (c + 1) * (NG // 2), group, 0)
    plsc.subcore_barrier()
    pltpu.sync_copy(acc.at[pl.ds(s * RPT, RPT)],
                    out_hbm.at[pl.ds(c * NP + s * RPT, RPT)])


@functools.partial(
    pl.kernel,
    out_type=jax.ShapeDtypeStruct((B * NP, F), jnp.float32),
    mesh=_mesh,
    scratch_types=[
        pltpu.VMEM_SHARED((NP, F), jnp.float32),
        pltpu.VMEM((G, CHUNK), jnp.int32),
        pltpu.VMEM((G, CHUNK), jnp.int32),
        pltpu.VMEM((CHUNK, F), jnp.float32),
        pltpu.SemaphoreType.DMA,
    ],
)
def _agg_kernel(z_hbm, srcg_hbm, dstl_hbm, zeros_hbm, out_hbm,
                acc, sidx, didx, rows, sem):
    c = lax.axis_index("c")
    s = lax.axis_index("s")
    for bi in range(2):
        b = 2 * c + bi
        pltpu.sync_copy(zeros_hbm, acc.at[pl.ds(s * RPT, RPT)])
        plsc.subcore_barrier()

        def group(g, carry):
            pltpu.sync_copy(srcg_hbm.at[b, s, pl.ds(g * G, G)], sidx)
            pltpu.sync_copy(dstl_hbm.at[s, pl.ds(g * G, G)], didx)

            def chunk(k, carry2):
                pltpu.async_copy(z_hbm.at[sidx.at[k]], rows, sem).wait()
                pltpu.sync_copy(rows, acc.at[didx.at[k]], add=True)
                return carry2

            return lax.fori_loop(0, G, chunk, carry)

        lax.fori_loop(0, NG, group, 0)
        plsc.subcore_barrier()
        pltpu.sync_copy(acc.at[pl.ds(s * RPT, RPT)],
                        out_hbm.at[pl.ds(b * NP + s * RPT, RPT)])


_BN = 2560


def _dinv_of(dinv_blk):
    return dinv_blk[:, 0:1]


def _tc_dinv_body(da_ref, db_ref, dinv_ref):
    deg = da_ref[:, 0:1] + db_ref[:, 0:1] + 1.0
    dinv_ref[...] = jnp.broadcast_to(lax.rsqrt(deg), dinv_ref.shape)


def _tc_dinv(da, db):
    return pl.pallas_call(
        _tc_dinv_body,
        grid=(NP // _BN,),
        in_specs=[pl.BlockSpec((_BN, F), lambda n: (n, 0)),
                  pl.BlockSpec((_BN, F), lambda n: (n, 0))],
        out_specs=pl.BlockSpec((_BN, 16), lambda n: (n, 0)),
        out_shape=jax.ShapeDtypeStruct((NP, 16), jnp.float32),
    )(da, db)


def _tc_z1_body(x_ref, w1_ref, deg_ref, z1_ref):
    dinv = _dinv_of(deg_ref[...])
    xw = lax.dot_general(x_ref[0], w1_ref[...], (((1,), (1,)), ((), ())),
                         preferred_element_type=jnp.float32,
                         precision=lax.Precision.HIGHEST)
    z1_ref[0] = xw * dinv


def _tc_mid_body(p1_ref, z1_ref, deg_ref, b1_ref, w2_ref, z2_ref):
    dinv = _dinv_of(deg_ref[...])
    h = dinv * (p1_ref[0] + z1_ref[0]) + b1_ref[...]
    h = 0.5 * h * (1.0 + lax.erf(h * (2.0 ** -0.5)))
    z2_ref[0] = lax.dot_general(h, w2_ref[...], (((1,), (1,)), ((), ())),
                                preferred_element_type=jnp.float32,
                                precision=lax.Precision.HIGHEST) * dinv


def _tc_out_body(p2_ref, z2_ref, deg_ref, b2_ref, out_ref):
    dinv = _dinv_of(deg_ref[...])
    out_ref[0] = dinv * (p2_ref[0] + z2_ref[0]) + b2_ref[...]


def _node_spec():
    return pl.BlockSpec((1, _BN, F), lambda b, n: (b, n, 0))


def _deg_spec():
    return pl.BlockSpec((_BN, 16), lambda b, n: (n, 0))


def _full_spec(shape):
    return pl.BlockSpec(shape, lambda b, n: tuple(0 for _ in shape))


_grid = (B, NP // _BN)


def _tc_z1(x_pad, w1, deg):
    return pl.pallas_call(
        _tc_z1_body,
        grid=_grid,
        in_specs=[_node_spec(), _full_spec((F, F)), _deg_spec()],
        out_specs=_node_spec(),
        out_shape=jax.ShapeDtypeStruct((B, NP, F), jnp.float32),
    )(x_pad, w1, deg)


def _tc_mid(p1, z1, deg, b1, w2):
    return pl.pallas_call(
        _tc_mid_body,
        grid=_grid,
        in_specs=[_node_spec(), _node_spec(), _deg_spec(),
                  _full_spec((1, F)), _full_spec((F, F))],
        out_specs=_node_spec(),
        out_shape=jax.ShapeDtypeStruct((B, NP, F), jnp.float32),
    )(p1, z1, deg, b1, w2)


def _tc_out(p2, z2, deg, b2):
    return pl.pallas_call(
        _tc_out_body,
        grid=_grid,
        in_specs=[_node_spec(), _node_spec(), _deg_spec(), _full_spec((1, F))],
        out_specs=_node_spec(),
        out_shape=jax.ShapeDtypeStruct((B, NP, F), jnp.float32),
    )(p2, z2, deg, b2)


def kernel(x, edge_index, W1, b1, W2, b2):
    src = edge_index[0].astype(jnp.int32)
    dst = edge_index[1].astype(jnp.int32)
    pad = jnp.full((EP - E,), N, dtype=jnp.int32)
    src = jnp.concatenate([src, pad])
    dst = jnp.concatenate([dst, pad])
    dstl = dst.reshape(NSUB, K, CHUNK)
    srcg = (src[None, :] +
            (jnp.arange(B, dtype=jnp.int32) * NP)[:, None]).reshape(
                B, NSUB, K, CHUNK)

    x_pad = jnp.pad(x, ((0, 0), (0, NP - N), (0, 0)))
    onesF = jnp.ones((CHUNK, F), jnp.float32)
    zerosF = jnp.zeros((RPT, F), jnp.float32)

    deg2 = _deg_kernel(dstl, onesF, zerosF)
    deg = _tc_dinv(deg2[:NP], deg2[NP:])

    z1 = _tc_z1(x_pad, W1, deg)
    p1 = _agg_kernel(z1.reshape(B * NP, F), srcg, dstl, zerosF)
    z2 = _tc_mid(p1.reshape(B, NP, F), z1, deg, b1.reshape(1, F), W2)
    p2 = _agg_kernel(z2.reshape(B * NP, F), srcg, dstl, zerosF)
    out = _tc_out(p2.reshape(B, NP, F), z2, deg, b2.reshape(1, F))
    return out[:, :N, :]


# speedup vs baseline: 12.8408x; 1.1780x over previous
import functools

import jax
import jax.numpy as jnp
from jax import lax
from jax.experimental import pallas as pl
from jax.experimental.pallas import tpu as pltpu
from jax.experimental.pallas import tpu_sc as plsc

N = 10000
NP = 10240
B = 4
F = 128
E = 320000
CHUNK = 128
NSUB = 16
G = 16
NG = 10
K = G * NG
EP = NSUB * CHUNK * K
RPT = NP // NSUB

_mesh = plsc.VectorSubcoreMesh(core_axis_name="c", subcore_axis_name="s",
                               num_cores=2, num_subcores=NSUB)


@functools.partial(
    pl.kernel,
    out_type=jax.ShapeDtypeStruct((2 * NP, F), jnp.float32),
    mesh=_mesh,
    scratch_types=[
        pltpu.VMEM_SHARED((NP, F), jnp.float32),
        pltpu.VMEM((G, CHUNK), jnp.int32),
        pltpu.VMEM((CHUNK, F), jnp.float32),
    ],
)
def _deg_kernel(dstl_hbm, ones_hbm, zeros_hbm, out_hbm, acc, didx, ones_v):
    c = lax.axis_index("c")
    s = lax.axis_index("s")
    pltpu.sync_copy(ones_hbm, ones_v)
    pltpu.sync_copy(zeros_hbm, acc.at[pl.ds(s * RPT, RPT)])
    plsc.subcore_barrier()

    def group(g, carry):
        pltpu.sync_copy(dstl_hbm.at[s, pl.ds(g * G, G)], didx)

        def chunk(k, carry2):
            pltpu.sync_copy(ones_v, acc.at[didx.at[k]], add=True)
            return carry2

        return lax.fori_loop(0, G, chunk, carry)

    lax.fori_loop(c * (NG // 2), (c + 1) * (NG // 2), group, 0)
    plsc.subcore_barrier()
    pltpu.sync_copy(acc.at[pl.ds(s * RPT, RPT)],
                    out_hbm.at[pl.ds(c * NP + s * RPT, RPT)])


@functools.partial(
    pl.kernel,
    out_type=jax.ShapeDtypeStruct((B * NP, F), jnp.float32),
    mesh=_mesh,
    scratch_types=[
        pltpu.VMEM_SHARED((NP, F), jnp.float32),
        pltpu.VMEM((G, CHUNK), jnp.int32),
        pltpu.VMEM((G, CHUNK), jnp.int32),
        pltpu.VMEM((CHUNK, F), jnp.float32),
        pltpu.VMEM((CHUNK, F), jnp.float32),
        pltpu.SemaphoreType.DMA,
        pltpu.SemaphoreType.DMA,
    ],
)
def _agg_kernel(z_hbm, srcg_hbm, dstl_hbm, zeros_hbm, out_hbm,
                acc, sidx, didx, rows0, rows1, sem0, sem1):
    c = lax.axis_index("c")
    s = lax.axis_index("s")
    rows = (rows0, rows1)
    sems = (sem0, sem1)
    for bi in range(2):
        b = 2 * c + bi
        pltpu.sync_copy(zeros_hbm, acc.at[pl.ds(s * RPT, RPT)])
        plsc.subcore_barrier()

        for g in range(NG):
            pltpu.sync_copy(srcg_hbm.at[b, s, pl.ds(g * G, G)], sidx)
            pltpu.sync_copy(dstl_hbm.at[s, pl.ds(g * G, G)], didx)
            pltpu.async_copy(z_hbm.at[sidx.at[0]], rows0, sem0)
            pltpu.async_copy(z_hbm.at[sidx.at[1]], rows1, sem1)

            def pair(kk, carry2, _b=b):
                for j in range(2):
                    k = 2 * kk + j
                    pltpu.make_async_copy(z_hbm.at[pl.ds(0, CHUNK)],
                                          rows[j], sems[j]).wait()
                    pltpu.sync_copy(rows[j], acc.at[didx.at[k]], add=True)

                    @pl.when(k + 2 < G)
                    def _():
                        pltpu.async_copy(z_hbm.at[sidx.at[k + 2]],
                                         rows[j], sems[j])
                return carry2

            lax.fori_loop(0, G // 2, pair, 0)
        plsc.subcore_barrier()
        pltpu.sync_copy(acc.at[pl.ds(s * RPT, RPT)],
                        out_hbm.at[pl.ds(b * NP + s * RPT, RPT)])


_BN = 2560


def _dinv_of(dinv_blk):
    return dinv_blk[:, 0:1]


def _tc_dinv_body(da_ref, db_ref, dinv_ref):
    deg = da_ref[:, 0:1] + db_ref[:, 0:1] + 1.0
    dinv_ref[...] = jnp.broadcast_to(lax.rsqrt(deg), dinv_ref.shape)


def _tc_dinv(da, db):
    return pl.pallas_call(
        _tc_dinv_body,
        grid=(NP // _BN,),
        in_specs=[pl.BlockSpec((_BN, F), lambda n: (n, 0)),
                  pl.BlockSpec((_BN, F), lambda n: (n, 0))],
        out_specs=pl.BlockSpec((_BN, 16), lambda n: (n, 0)),
        out_shape=jax.ShapeDtypeStruct((NP, 16), jnp.float32),
    )(da, db)


def _tc_z1_body(x_ref, w1_ref, deg_ref, z1_ref):
    dinv = _dinv_of(deg_ref[...])
    xw = lax.dot_general(x_ref[0], w1_ref[...], (((1,), (1,)), ((), ())),
                         preferred_element_type=jnp.float32,
                         precision=lax.Precision.HIGHEST)
    z1_ref[0] = xw * dinv


def _tc_mid_body(p1_ref, z1_ref, deg_ref, b1_ref, w2_ref, z2_ref):
    dinv = _dinv_of(deg_ref[...])
    h = dinv * (p1_ref[0] + z1_ref[0]) + b1_ref[...]
    h = 0.5 * h * (1.0 + lax.erf(h * (2.0 ** -0.5)))
    z2_ref[0] = lax.dot_general(h, w2_ref[...], (((1,), (1,)), ((), ())),
                                preferred_element_type=jnp.float32,
                                precision=lax.Precision.HIGHEST) * dinv


def _tc_out_body(p2_ref, z2_ref, deg_ref, b2_ref, out_ref):
    dinv = _dinv_of(deg_ref[...])
    out_ref[0] = dinv * (p2_ref[0] + z2_ref[0]) + b2_ref[...]


def _node_spec():
    return pl.BlockSpec((1, _BN, F), lambda b, n: (b, n, 0))


def _deg_spec():
    return pl.BlockSpec((_BN, 16), lambda b, n: (n, 0))


def _full_spec(shape):
    return pl.BlockSpec(shape, lambda b, n: tuple(0 for _ in shape))


_grid = (B, NP // _BN)


def _tc_z1(x_pad, w1, deg):
    return pl.pallas_call(
        _tc_z1_body,
        grid=_grid,
        in_specs=[_node_spec(), _full_spec((F, F)), _deg_spec()],
        out_specs=_node_spec(),
        out_shape=jax.ShapeDtypeStruct((B, NP, F), jnp.float32),
    )(x_pad, w1, deg)


def _tc_mid(p1, z1, deg, b1, w2):
    return pl.pallas_call(
        _tc_mid_body,
        grid=_grid,
        in_specs=[_node_spec(), _node_spec(), _deg_spec(),
                  _full_spec((1, F)), _full_spec((F, F))],
        out_specs=_node_spec(),
        out_shape=jax.ShapeDtypeStruct((B, NP, F), jnp.float32),
    )(p1, z1, deg, b1, w2)


def _tc_out(p2, z2, deg, b2):
    return pl.pallas_call(
        _tc_out_body,
        grid=_grid,
        in_specs=[_node_spec(), _node_spec(), _deg_spec(), _full_spec((1, F))],
        out_specs=_node_spec(),
        out_shape=jax.ShapeDtypeStruct((B, NP, F), jnp.float32),
    )(p2, z2, deg, b2)


def kernel(x, edge_index, W1, b1, W2, b2):
    src = edge_index[0].astype(jnp.int32)
    dst = edge_index[1].astype(jnp.int32)
    pad = jnp.full((EP - E,), N, dtype=jnp.int32)
    src = jnp.concatenate([src, pad])
    dst = jnp.concatenate([dst, pad])
    dstl = dst.reshape(NSUB, K, CHUNK)
    srcg = (src[None, :] +
            (jnp.arange(B, dtype=jnp.int32) * NP)[:, None]).reshape(
                B, NSUB, K, CHUNK)

    x_pad = jnp.pad(x, ((0, 0), (0, NP - N), (0, 0)))
    onesF = jnp.ones((CHUNK, F), jnp.float32)
    zerosF = jnp.zeros((RPT, F), jnp.float32)

    deg2 = _deg_kernel(dstl, onesF, zerosF)
    deg = _tc_dinv(deg2[:NP], deg2[NP:])

    z1 = _tc_z1(x_pad, W1, deg)
    p1 = _agg_kernel(z1.reshape(B * NP, F), srcg, dstl, zerosF)
    z2 = _tc_mid(p1.reshape(B, NP, F), z1, deg, b1.reshape(1, F), W2)
    p2 = _agg_kernel(z2.reshape(B * NP, F), srcg, dstl, zerosF)
    out = _tc_out(p2.reshape(B, NP, F), z2, deg, b2.reshape(1, F))
    return out[:, :N, :]


# speedup vs baseline: 21.4781x; 1.6726x over previous
import functools

import jax
import jax.numpy as jnp
from jax import lax
from jax.experimental import pallas as pl
from jax.experimental.pallas import tpu as pltpu
from jax.experimental.pallas import tpu_sc as plsc

N = 10000
NP = 10240
B = 4
F = 128
E = 320000
CHUNK = 112
NSUB = 16
G = 18
NG = 10
K = G * NG
EP = NSUB * CHUNK * K
RPT = NP // NSUB

_mesh = plsc.VectorSubcoreMesh(core_axis_name="c", subcore_axis_name="s",
                               num_cores=2, num_subcores=NSUB)


@functools.partial(
    pl.kernel,
    out_type=jax.ShapeDtypeStruct((2 * NP, F), jnp.float32),
    mesh=_mesh,
    scratch_types=[
        pltpu.VMEM_SHARED((NP, F), jnp.float32),
        pltpu.VMEM((G, CHUNK), jnp.int32),
        pltpu.VMEM((CHUNK, F), jnp.float32),
    ],
)
def _deg_kernel(dstl_hbm, ones_hbm, zeros_hbm, out_hbm, acc, didx, ones_v):
    c = lax.axis_index("c")
    s = lax.axis_index("s")
    pltpu.sync_copy(ones_hbm, ones_v)
    pltpu.sync_copy(zeros_hbm, acc.at[pl.ds(s * RPT, RPT)])
    plsc.subcore_barrier()

    def group(g, carry):
        pltpu.sync_copy(dstl_hbm.at[s, g], didx)

        def chunk(k, carry2):
            pltpu.sync_copy(ones_v, acc.at[didx.at[k]], add=True)
            return carry2

        return lax.fori_loop(0, G, chunk, carry)

    lax.fori_loop(c * (NG // 2), (c + 1) * (NG // 2), group, 0)
    plsc.subcore_barrier()
    pltpu.sync_copy(acc.at[pl.ds(s * RPT, RPT)],
                    out_hbm.at[pl.ds(c * NP + s * RPT, RPT)])


@functools.partial(
    pl.kernel,
    out_type=jax.ShapeDtypeStruct((B * NP, F), jnp.float32),
    mesh=_mesh,
    scratch_types=[
        pltpu.VMEM_SHARED((NP, F), jnp.float32),
        pltpu.VMEM((G, CHUNK), jnp.int32),
        pltpu.VMEM((G, CHUNK), jnp.int32),
        pltpu.VMEM((CHUNK, F), jnp.float32),
        pltpu.VMEM((CHUNK, F), jnp.float32),
        pltpu.VMEM((CHUNK, F), jnp.float32),
        pltpu.SemaphoreType.DMA,
        pltpu.SemaphoreType.DMA,
        pltpu.SemaphoreType.DMA,
        pltpu.SemaphoreType.DMA,
        pltpu.SemaphoreType.DMA,
        pltpu.SemaphoreType.DMA,
    ],
)
def _agg_kernel(z_hbm, srcg_hbm, dstl_hbm, zeros_hbm, out_hbm,
                acc, sidx, didx, rows0, rows1, rows2,
                sg0, sg1, sg2, ss0, ss1, ss2):
    c = lax.axis_index("c")
    s = lax.axis_index("s")
    rows = (rows0, rows1, rows2)
    sg = (sg0, sg1, sg2)
    ss = (ss0, ss1, ss2)

    def wait_scatter(j):
        pltpu.make_async_copy(rows[j], acc.at[pl.ds(0, CHUNK)], ss[j]).wait()

    def wait_gather(j):
        pltpu.make_async_copy(z_hbm.at[pl.ds(0, CHUNK)], rows[j], sg[j]).wait()

    for bi in range(2):
        b = 2 * c + bi
        pltpu.sync_copy(zeros_hbm, acc.at[pl.ds(s * RPT, RPT)])
        plsc.subcore_barrier()

        def group(g, carry, _b=b):
            pltpu.sync_copy(srcg_hbm.at[_b, s, g], sidx)
            pltpu.sync_copy(dstl_hbm.at[s, g], didx)
            pltpu.async_copy(z_hbm.at[sidx.at[0]], rows0, sg0)
            pltpu.async_copy(z_hbm.at[sidx.at[1]], rows1, sg1)

            def triple(kk, carry2):
                for j in range(3):
                    k = 3 * kk + j
                    jp = (j + 2) % 3
                    wait_gather(j)
                    pltpu.async_copy(rows[j], acc.at[didx.at[k]], ss[j],
                                     add=True)
                    if j == 0:
                        @pl.when(kk > 0)
                        def _():
                            wait_scatter(2)
                    else:
                        wait_scatter(j - 1)

                    @pl.when(k + 2 < G)
                    def _():
                        pltpu.async_copy(z_hbm.at[sidx.at[k + 2]],
                                         rows[jp], sg[jp])
                return carry2

            lax.fori_loop(0, G // 3, triple, 0)
            wait_scatter((G - 1) % 3)
            return carry

        lax.fori_loop(0, NG, group, 0)
        plsc.subcore_barrier()
        pltpu.sync_copy(acc.at[pl.ds(s * RPT, RPT)],
                        out_hbm.at[pl.ds(b * NP + s * RPT, RPT)])


_BN = 2560


def _dinv_of(dinv_blk):
    return dinv_blk[:, 0:1]


def _tc_dinv_body(da_ref, db_ref, dinv_ref):
    deg = da_ref[:, 0:1] + db_ref[:, 0:1] + 1.0
    dinv_ref[...] = jnp.broadcast_to(lax.rsqrt(deg), dinv_ref.shape)


def _tc_dinv(da, db):
    return pl.pallas_call(
        _tc_dinv_body,
        grid=(NP // _BN,),
        in_specs=[pl.BlockSpec((_BN, F), lambda n: (n, 0)),
                  pl.BlockSpec((_BN, F), lambda n: (n, 0))],
        out_specs=pl.BlockSpec((_BN, 16), lambda n: (n, 0)),
        out_shape=jax.ShapeDtypeStruct((NP, 16), jnp.float32),
    )(da, db)


def _tc_z1_body(x_ref, w1_ref, deg_ref, z1_ref):
    dinv = _dinv_of(deg_ref[...])
    xw = lax.dot_general(x_ref[0], w1_ref[...], (((1,), (1,)), ((), ())),
                         preferred_element_type=jnp.float32,
                         precision=lax.Precision.HIGHEST)
    z1_ref[0] = xw * dinv


def _tc_mid_body(p1_ref, z1_ref, deg_ref, b1_ref, w2_ref, z2_ref):
    dinv = _dinv_of(deg_ref[...])
    h = dinv * (p1_ref[0] + z1_ref[0]) + b1_ref[...]
    h = 0.5 * h * (1.0 + lax.erf(h * (2.0 ** -0.5)))
    z2_ref[0] = lax.dot_general(h, w2_ref[...], (((1,), (1,)), ((), ())),
                                preferred_element_type=jnp.float32,
                                precision=lax.Precision.HIGHEST) * dinv


def _tc_out_body(p2_ref, z2_ref, deg_ref, b2_ref, out_ref):
    dinv = _dinv_of(deg_ref[...])
    out_ref[0] = dinv * (p2_ref[0] + z2_ref[0]) + b2_ref[...]


def _node_spec():
    return pl.BlockSpec((1, _BN, F), lambda b, n: (b, n, 0))


def _deg_spec():
    return pl.BlockSpec((_BN, 16), lambda b, n: (n, 0))


def _full_spec(shape):
    return pl.BlockSpec(shape, lambda b, n: tuple(0 for _ in shape))


_grid = (B, NP // _BN)


def _tc_z1(x_pad, w1, deg):
    return pl.pallas_call(
        _tc_z1_body,
        grid=_grid,
        in_specs=[_node_spec(), _full_spec((F, F)), _deg_spec()],
        out_specs=_node_spec(),
        out_shape=jax.ShapeDtypeStruct((B, NP, F), jnp.float32),
    )(x_pad, w1, deg)


def _tc_mid(p1, z1, deg, b1, w2):
    return pl.pallas_call(
        _tc_mid_body,
        grid=_grid,
        in_specs=[_node_spec(), _node_spec(), _deg_spec(),
                  _full_spec((1, F)), _full_spec((F, F))],
        out_specs=_node_spec(),
        out_shape=jax.ShapeDtypeStruct((B, NP, F), jnp.float32),
    )(p1, z1, deg, b1, w2)


def _tc_out(p2, z2, deg, b2):
    return pl.pallas_call(
        _tc_out_body,
        grid=_grid,
        in_specs=[_node_spec(), _node_spec(), _deg_spec(), _full_spec((1, F))],
        out_specs=_node_spec(),
        out_shape=jax.ShapeDtypeStruct((B, NP, F), jnp.float32),
    )(p2, z2, deg, b2)


def kernel(x, edge_index, W1, b1, W2, b2):
    src = edge_index[0].astype(jnp.int32)
    dst = edge_index[1].astype(jnp.int32)
    pad = jnp.full((EP - E,), N, dtype=jnp.int32)
    src = jnp.concatenate([src, pad])
    dst = jnp.concatenate([dst, pad])
    dstl = dst.reshape(NSUB, NG, G, CHUNK)
    srcg = (src[None, :] +
            (jnp.arange(B, dtype=jnp.int32) * NP)[:, None]).reshape(
                B, NSUB, NG, G, CHUNK)

    x_pad = jnp.pad(x, ((0, 0), (0, NP - N), (0, 0)))
    onesF = jnp.ones((CHUNK, F), jnp.float32)
    zerosF = jnp.zeros((RPT, F), jnp.float32)

    deg2 = _deg_kernel(dstl, onesF, zerosF)
    deg = _tc_dinv(deg2[:NP], deg2[NP:])

    z1 = _tc_z1(x_pad, W1, deg)
    p1 = _agg_kernel(z1.reshape(B * NP, F), srcg, dstl, zerosF)
    z2 = _tc_mid(p1.reshape(B, NP, F), z1, deg, b1.reshape(1, F), W2)
    p2 = _agg_kernel(z2.reshape(B * NP, F), srcg, dstl, zerosF)
    out = _tc_out(p2.reshape(B, NP, F), z2, deg, b2.reshape(1, F))
    return out[:, :N, :]
